# baseline (speedup 1.0000x reference)
import jax
import jax.numpy as jnp
from jax import lax
from jax.experimental import pallas as pl
from jax.experimental.pallas import tpu as pltpu

_TILE = 12800
_TB = 64
_DN = (((0,), (0,)), ((), ()))


def _body(cont_ref, code_ref, w_ref, b_ref, pt_ref, ct_ref, out_ref):
    c3 = cont_ref[...]
    c = c3.reshape(_TILE, c3.shape[2]).astype(jnp.bfloat16)
    z = jnp.dot(c, w_ref[...], preferred_element_type=jnp.float32) + b_ref[...]
    n = c.shape[0]
    code = code_ref[...].astype(jnp.int32)
    pdg = jnp.bitwise_and(code, 7)
    chg = jnp.right_shift(code, 3)
    oh_p = (lax.broadcasted_iota(jnp.int32, (8, n), 0) == pdg).astype(jnp.bfloat16)
    oh_c = (lax.broadcasted_iota(jnp.int32, (4, n), 0) == chg).astype(jnp.bfloat16)
    z_p = lax.dot_general(oh_p, pt_ref[...], _DN, preferred_element_type=jnp.float32)
    z_c = lax.dot_general(oh_c, ct_ref[...], _DN, preferred_element_type=jnp.float32)
    out_ref[...] = jnp.concatenate([z, z_p, z_c], axis=1)


def kernel(cont, pdgid, charge, W, b, pdgid_table, charge_table):
    Bb, L, F = cont.shape
    rows = Bb * L
    code1 = (pdgid | (charge << 3)).astype(jnp.int8).reshape(1, rows)
    out_dim = W.shape[0] + pdgid_table.shape[1] + charge_table.shape[1]
    grid = rows // _TILE

    out = pl.pallas_call(
        _body,
        grid=(grid,),
        in_specs=[
            pl.BlockSpec((_TB, L, F), lambda i: (i, 0, 0)),
            pl.BlockSpec((1, _TILE), lambda i: (0, i)),
            pl.BlockSpec((F, W.shape[0]), lambda i: (0, 0)),
            pl.BlockSpec((1, W.shape[0]), lambda i: (0, 0)),
            pl.BlockSpec(pdgid_table.shape, lambda i: (0, 0)),
            pl.BlockSpec(charge_table.shape, lambda i: (0, 0)),
        ],
        out_specs=pl.BlockSpec((_TILE, out_dim), lambda i: (i, 0)),
        out_shape=jax.ShapeDtypeStruct((rows, out_dim), jnp.float32),
        compiler_params=pltpu.CompilerParams(
            dimension_semantics=("arbitrary",),
        ),
    )(cont, code1, W.T.astype(jnp.bfloat16),
      b.reshape(1, -1), pdgid_table.astype(jnp.bfloat16),
      charge_table.astype(jnp.bfloat16))
    return out.reshape(Bb, L, out_dim)


# speedup vs baseline: 1.3174x; 1.3174x over previous
import jax
import jax.numpy as jnp
from jax import lax
from jax.experimental import pallas as pl
from jax.experimental.pallas import tpu as pltpu

_TILE = 16384
_DN = (((0,), (0,)), ((), ()))


def _body(cont_ref, code_ref, w_ref, b_ref, pt_ref, ct_ref, out_ref):
    c = cont_ref[...].astype(jnp.bfloat16)
    z = jnp.dot(c, w_ref[...], preferred_element_type=jnp.float32) + b_ref[...]
    n = c.shape[0]
    code = code_ref[...].astype(jnp.int32)
    pdg = jnp.bitwise_and(code, 7)
    chg = jnp.right_shift(code, 3)
    oh_p = (lax.broadcasted_iota(jnp.int32, (8, n), 0) == pdg).astype(jnp.bfloat16)
    oh_c = (lax.broadcasted_iota(jnp.int32, (4, n), 0) == chg).astype(jnp.bfloat16)
    z_p = lax.dot_general(oh_p, pt_ref[...], _DN, preferred_element_type=jnp.float32)
    z_c = lax.dot_general(oh_c, ct_ref[...], _DN, preferred_element_type=jnp.float32)
    out_ref[...] = jnp.concatenate([z, z_p, z_c], axis=1)


def kernel(cont, pdgid, charge, W, b, pdgid_table, charge_table):
    Bb, L, F = cont.shape
    rows = Bb * L
    cont2 = cont.reshape(rows, F)
    code1 = (pdgid | (charge << 3)).astype(jnp.int8).reshape(1, rows)
    out_dim = W.shape[0] + pdgid_table.shape[1] + charge_table.shape[1]
    grid = rows // _TILE

    out = pl.pallas_call(
        _body,
        grid=(grid,),
        in_specs=[
            pl.BlockSpec((_TILE, F), lambda i: (i, 0)),
            pl.BlockSpec((1, _TILE), lambda i: (0, i)),
            pl.BlockSpec((F, W.shape[0]), lambda i: (0, 0)),
            pl.BlockSpec((1, W.shape[0]), lambda i: (0, 0)),
            pl.BlockSpec(pdgid_table.shape, lambda i: (0, 0)),
            pl.BlockSpec(charge_table.shape, lambda i: (0, 0)),
        ],
        out_specs=pl.BlockSpec((_TILE, out_dim), lambda i: (i, 0)),
        out_shape=jax.ShapeDtypeStruct((rows, out_dim), jnp.float32),
        compiler_params=pltpu.CompilerParams(
            dimension_semantics=("arbitrary",),
        ),
    )(cont2, code1, W.T.astype(jnp.bfloat16),
      b.reshape(1, -1), pdgid_table.astype(jnp.bfloat16),
      charge_table.astype(jnp.bfloat16))
    return out.reshape(Bb, L, out_dim)
